# baseline (speedup 1.0000x reference)
import jax
import jax.numpy as jnp
from jax import lax
from jax.experimental import pallas as pl
from jax.experimental.pallas import tpu as pltpu
from jax.experimental.pallas import tpu_sc as plsc

N = 10000
E = 160000
D = 256
HALF = 128
CHUNK = 80
CHUNKS_PER_TILE = E // CHUNK // 16
ROW_CHUNKS = N // CHUNK
N_HIDDEN = 3
INV_SQRT2 = 0.7071067811865475
NBUF = 3


def _silu(v):
    return v / (1.0 + jnp.exp(-v))


def _edge_body(rbf_ref, ea_ref, wro_ref, wdr_ref, xm_ref):
    emb = jnp.dot(rbf_ref[...], wro_ref[...], preferred_element_type=jnp.float32)
    emb = jnp.dot(emb, wdr_ref[...], preferred_element_type=jnp.float32)
    prod = ea_ref[...] * emb
    xm_ref[0] = prod[:, :HALF]
    xm_ref[1] = prod[:, HALF:]


def _edge_stage(rbf, edge_attr, W_rbf_out, W_dense_rbf):
    blk = 1280
    grid = E // blk
    return pl.pallas_call(
        _edge_body,
        grid=(grid,),
        in_specs=[
            pl.BlockSpec((blk, 16), lambda i: (i, 0)),
            pl.BlockSpec((blk, D), lambda i: (i, 0)),
            pl.BlockSpec((16, 16), lambda i: (0, 0)),
            pl.BlockSpec((16, D), lambda i: (0, 0)),
        ],
        out_specs=pl.BlockSpec((2, blk, HALF), lambda i: (0, i, 0)),
        out_shape=jax.ShapeDtypeStruct((2, E, HALF), jnp.float32),
    )(rbf, edge_attr, W_rbf_out, W_dense_rbf)


def _scatter_body(xm_hbm, dst_hbm, out_hbm, dst_v, xm_buf, accum, sems):
    c = lax.axis_index("c")
    s = lax.axis_index("s")

    zero_buf = xm_buf.at[0]
    zeros16 = jnp.zeros((16,), jnp.float32)

    def _zrow(i, carry):
        for j in range(HALF // 16):
            zero_buf[i, pl.ds(j * 16, 16)] = zeros16
        return carry

    lax.fori_loop(0, CHUNK, _zrow, 0)

    for k in range(8):
        idx = s + 16 * k

        @pl.when(idx < ROW_CHUNKS)
        def _():
            pltpu.sync_copy(zero_buf, accum.at[pl.ds(idx * CHUNK, CHUNK)])

    plsc.subcore_barrier()

    pltpu.sync_copy(dst_hbm.at[s], dst_v)

    def _load(j, b):
        g = s * CHUNKS_PER_TILE + j
        row0 = pl.multiple_of(g * CHUNK, CHUNK)
        return pltpu.make_async_copy(
            xm_hbm.at[c, pl.ds(row0, CHUNK)], xm_buf.at[b], sems.at[b])

    for b in range(NBUF):
        _load(b, b).start()

    def _hyper(g, carry):
        for b in range(NBUF):
            j = g * NBUF + b

            @pl.when(j < CHUNKS_PER_TILE)
            def _():
                _load(j, b).wait()
                pltpu.sync_copy(xm_buf.at[b], accum.at[dst_v.at[j]], add=True)

                @pl.when(j + NBUF < CHUNKS_PER_TILE)
                def _():
                    _load(j + NBUF, b).start()
        return carry

    lax.fori_loop(0, (CHUNKS_PER_TILE + NBUF - 1) // NBUF, _hyper, 0)

    plsc.subcore_barrier()

    for k in range(8):
        idx = s + 16 * k

        @pl.when(idx < ROW_CHUNKS)
        def _():
            row0 = pl.multiple_of(idx * CHUNK, CHUNK)
            pltpu.sync_copy(
                accum.at[pl.ds(row0, CHUNK)],
                out_hbm.at[c, pl.ds(row0, CHUNK)],
            )


def _scatter_stage(xm, dst3d):
    mesh = plsc.VectorSubcoreMesh(core_axis_name="c", subcore_axis_name="s")
    f = pl.kernel(
        _scatter_body,
        out_type=jax.ShapeDtypeStruct((2, N, HALF), jnp.float32),
        mesh=mesh,
        scratch_types=[
            pltpu.VMEM((CHUNKS_PER_TILE, CHUNK), jnp.int32),
            pltpu.VMEM((NBUF, CHUNK, HALF), jnp.float32),
            pltpu.VMEM_SHARED((N, HALF), jnp.float32),
            pltpu.SemaphoreType.DMA((NBUF,)),
        ],
    )
    return f(xm, dst3d)


def _mlp_body(xe_ref, w1_ref, wres_ref, wout_ref, out_ref):
    z = (jnp.dot(xe_ref[0], w1_ref[:HALF, :], preferred_element_type=jnp.float32)
         + jnp.dot(xe_ref[1], w1_ref[HALF:, :], preferred_element_type=jnp.float32))
    h = _silu(z)
    for i in range(N_HIDDEN):
        t = _silu(jnp.dot(h, wres_ref[i, 0], preferred_element_type=jnp.float32))
        t = _silu(jnp.dot(t, wres_ref[i, 1], preferred_element_type=jnp.float32))
        h = (h + t) * INV_SQRT2
    out_ref[...] = jnp.dot(h, wout_ref[...], preferred_element_type=jnp.float32)


def _mlp_stage(xe, W1, W_res, W_out):
    blk = 1000
    grid = N // blk
    return pl.pallas_call(
        _mlp_body,
        grid=(grid,),
        in_specs=[
            pl.BlockSpec((2, blk, HALF), lambda i: (0, i, 0)),
            pl.BlockSpec((D, D), lambda i: (0, 0)),
            pl.BlockSpec((N_HIDDEN, 2, D, D), lambda i: (0, 0, 0, 0)),
            pl.BlockSpec((D, 1), lambda i: (0, 0)),
        ],
        out_specs=pl.BlockSpec((blk, 1), lambda i: (i, 0)),
        out_shape=jax.ShapeDtypeStruct((N, 1), jnp.float32),
    )(xe, W1, W_res, W_out)


def kernel(x, edge_attr, edge_index, rbf, y, W_rbf_out, W_dense_rbf, W1, W_res, W_out):
    xm = _edge_stage(rbf, edge_attr, W_rbf_out, W_dense_rbf)
    dst3d = edge_index[1].reshape(16, CHUNKS_PER_TILE, CHUNK)
    xe = _scatter_stage(xm, dst3d)
    e_out = _mlp_stage(xe, W1, W_res, W_out)
    return (e_out, y)


# speedup vs baseline: 1.4080x; 1.4080x over previous
import jax
import jax.numpy as jnp
from jax import lax
from jax.experimental import pallas as pl
from jax.experimental.pallas import tpu as pltpu
from jax.experimental.pallas import tpu_sc as plsc

N = 10000
E = 160000
D = 256
HALF = 128
CHUNK = 80
CHUNKS_PER_TILE = E // CHUNK // 16
ROW_CHUNKS = N // CHUNK
N_HIDDEN = 3
INV_SQRT2 = 0.7071067811865475
NBUF = 3


def _silu(v):
    return v / (1.0 + jnp.exp(-v))


def _edge_body(rbf_t_ref, ea_ref, wro_ref, wdr_ref, xm_ref):
    wc = jnp.dot(wro_ref[...], wdr_ref[...], preferred_element_type=jnp.float32)
    emb = lax.dot_general(rbf_t_ref[...], wc, (((0,), (0,)), ((), ())),
                          preferred_element_type=jnp.float32)
    prod = ea_ref[...] * emb
    xm_ref[0] = prod[:, :HALF]
    xm_ref[1] = prod[:, HALF:]


def _edge_stage(rbf_t, edge_attr, W_rbf_out, W_dense_rbf):
    blk = 3200
    grid = E // blk
    return pl.pallas_call(
        _edge_body,
        grid=(grid,),
        in_specs=[
            pl.BlockSpec((16, blk), lambda i: (0, i)),
            pl.BlockSpec((blk, D), lambda i: (i, 0)),
            pl.BlockSpec((16, 16), lambda i: (0, 0)),
            pl.BlockSpec((16, D), lambda i: (0, 0)),
        ],
        out_specs=pl.BlockSpec((2, blk, HALF), lambda i: (0, i, 0)),
        out_shape=jax.ShapeDtypeStruct((2, E, HALF), jnp.float32),
    )(rbf_t, edge_attr, W_rbf_out, W_dense_rbf)


def _scatter_body(xm_hbm, dst_hbm, out_hbm, dst_v, xm_buf, accum, sems):
    c = lax.axis_index("c")
    s = lax.axis_index("s")

    zero_buf = xm_buf.at[0]
    zeros16 = jnp.zeros((16,), jnp.float32)

    def _zrow(i, carry):
        for j in range(HALF // 16):
            zero_buf[i, pl.ds(j * 16, 16)] = zeros16
        return carry

    lax.fori_loop(0, CHUNK, _zrow, 0)

    for k in range(8):
        idx = s + 16 * k

        @pl.when(idx < ROW_CHUNKS)
        def _():
            pltpu.sync_copy(zero_buf, accum.at[pl.ds(idx * CHUNK, CHUNK)])

    plsc.subcore_barrier()

    pltpu.sync_copy(dst_hbm.at[s], dst_v)

    def _load(j, b):
        g = s * CHUNKS_PER_TILE + j
        row0 = pl.multiple_of(g * CHUNK, CHUNK)
        return pltpu.make_async_copy(
            xm_hbm.at[c, pl.ds(row0, CHUNK)], xm_buf.at[b], sems.at[b])

    for b in range(NBUF):
        _load(b, b).start()

    def _hyper(g, carry):
        for b in range(NBUF):
            j = g * NBUF + b

            @pl.when(j < CHUNKS_PER_TILE)
            def _():
                _load(j, b).wait()
                pltpu.sync_copy(xm_buf.at[b], accum.at[dst_v.at[j]], add=True)

                @pl.when(j + NBUF < CHUNKS_PER_TILE)
                def _():
                    _load(j + NBUF, b).start()
        return carry

    lax.fori_loop(0, (CHUNKS_PER_TILE + NBUF - 1) // NBUF, _hyper, 0)

    plsc.subcore_barrier()

    for k in range(8):
        idx = s + 16 * k

        @pl.when(idx < ROW_CHUNKS)
        def _():
            row0 = pl.multiple_of(idx * CHUNK, CHUNK)
            pltpu.sync_copy(
                accum.at[pl.ds(row0, CHUNK)],
                out_hbm.at[c, pl.ds(row0, CHUNK)],
            )


def _scatter_stage(xm, dst3d):
    mesh = plsc.VectorSubcoreMesh(core_axis_name="c", subcore_axis_name="s")
    f = pl.kernel(
        _scatter_body,
        out_type=jax.ShapeDtypeStruct((2, N, HALF), jnp.float32),
        mesh=mesh,
        scratch_types=[
            pltpu.VMEM((CHUNKS_PER_TILE, CHUNK), jnp.int32),
            pltpu.VMEM((NBUF, CHUNK, HALF), jnp.float32),
            pltpu.VMEM_SHARED((N, HALF), jnp.float32),
            pltpu.SemaphoreType.DMA((NBUF,)),
        ],
    )
    return f(xm, dst3d)


def _mlp_body(xe_ref, w1_ref, wres_ref, wout_ref, out_ref):
    z = (jnp.dot(xe_ref[0], w1_ref[:HALF, :], preferred_element_type=jnp.float32)
         + jnp.dot(xe_ref[1], w1_ref[HALF:, :], preferred_element_type=jnp.float32))
    h = _silu(z)
    for i in range(N_HIDDEN):
        w0 = wres_ref[i, 0].astype(jnp.bfloat16)
        w1 = wres_ref[i, 1].astype(jnp.bfloat16)
        t = _silu(jnp.dot(h.astype(jnp.bfloat16), w0,
                          preferred_element_type=jnp.float32))
        t = _silu(jnp.dot(t.astype(jnp.bfloat16), w1,
                          preferred_element_type=jnp.float32))
        h = (h + t) * INV_SQRT2
    out_ref[...] = jnp.dot(h, wout_ref[...], preferred_element_type=jnp.float32)


def _mlp_stage(xe, W1, W_res, W_out):
    blk = 1000
    grid = N // blk
    return pl.pallas_call(
        _mlp_body,
        grid=(grid,),
        in_specs=[
            pl.BlockSpec((2, blk, HALF), lambda i: (0, i, 0)),
            pl.BlockSpec((D, D), lambda i: (0, 0)),
            pl.BlockSpec((N_HIDDEN, 2, D, D), lambda i: (0, 0, 0, 0)),
            pl.BlockSpec((D, 1), lambda i: (0, 0)),
        ],
        out_specs=pl.BlockSpec((blk, 1), lambda i: (i, 0)),
        out_shape=jax.ShapeDtypeStruct((N, 1), jnp.float32),
    )(xe, W1, W_res, W_out)


def kernel(x, edge_attr, edge_index, rbf, y, W_rbf_out, W_dense_rbf, W1, W_res, W_out):
    xm = _edge_stage(rbf.T, edge_attr, W_rbf_out, W_dense_rbf)
    dst3d = edge_index[1].reshape(16, CHUNKS_PER_TILE, CHUNK)
    xe = _scatter_stage(xm, dst3d)
    e_out = _mlp_stage(xe, W1, W_res, W_out)
    return (e_out, y)


# speedup vs baseline: 1.4336x; 1.0182x over previous
import jax
import jax.numpy as jnp
from jax import lax
from jax.experimental import pallas as pl
from jax.experimental.pallas import tpu as pltpu
from jax.experimental.pallas import tpu_sc as plsc

N = 10000
E = 160000
D = 256
HALF = 128
CHUNK = 80
CHUNKS_PER_TILE = E // CHUNK // 16
ROW_CHUNKS = N // CHUNK
N_HIDDEN = 3
INV_SQRT2 = 0.7071067811865475
NBUF = 3


def _silu(v):
    return v / (1.0 + jnp.exp(-v))


def _edge_body(rbf_t_ref, ea_ref, wro_ref, wdr_ref, xm_ref):
    wc = jnp.dot(wro_ref[...], wdr_ref[...], preferred_element_type=jnp.float32)
    emb = lax.dot_general(rbf_t_ref[...], wc, (((0,), (0,)), ((), ())),
                          preferred_element_type=jnp.float32)
    prod = ea_ref[...] * emb
    xm_ref[0] = prod[:, :HALF]
    xm_ref[1] = prod[:, HALF:]


def _edge_stage(rbf_t, edge_attr, W_rbf_out, W_dense_rbf):
    blk = 6400
    grid = E // blk
    return pl.pallas_call(
        _edge_body,
        grid=(grid,),
        in_specs=[
            pl.BlockSpec((16, blk), lambda i: (0, i)),
            pl.BlockSpec((blk, D), lambda i: (i, 0)),
            pl.BlockSpec((16, 16), lambda i: (0, 0)),
            pl.BlockSpec((16, D), lambda i: (0, 0)),
        ],
        out_specs=pl.BlockSpec((2, blk, HALF), lambda i: (0, i, 0)),
        out_shape=jax.ShapeDtypeStruct((2, E, HALF), jnp.float32),
    )(rbf_t, edge_attr, W_rbf_out, W_dense_rbf)


def _scatter_body(xm_hbm, dst_hbm, out_hbm, dst_v, xm_buf, accum, sems):
    c = lax.axis_index("c")
    s = lax.axis_index("s")

    zero_buf = xm_buf.at[0]
    zeros16 = jnp.zeros((16,), jnp.float32)

    def _zrow(i, carry):
        for j in range(HALF // 16):
            zero_buf[i, pl.ds(j * 16, 16)] = zeros16
        return carry

    lax.fori_loop(0, CHUNK, _zrow, 0)

    for k in range(8):
        idx = s + 16 * k

        @pl.when(idx < ROW_CHUNKS)
        def _():
            pltpu.sync_copy(zero_buf, accum.at[pl.ds(idx * CHUNK, CHUNK)])

    plsc.subcore_barrier()

    pltpu.sync_copy(dst_hbm.at[s], dst_v)

    def _load(j, b):
        g = s * CHUNKS_PER_TILE + j
        row0 = pl.multiple_of(g * CHUNK, CHUNK)
        return pltpu.make_async_copy(
            xm_hbm.at[c, pl.ds(row0, CHUNK)], xm_buf.at[b], sems.at[b])

    for b in range(NBUF):
        _load(b, b).start()

    def _hyper(g, carry):
        for b in range(NBUF):
            j = g * NBUF + b

            @pl.when(j < CHUNKS_PER_TILE)
            def _():
                _load(j, b).wait()
                pltpu.sync_copy(xm_buf.at[b], accum.at[dst_v.at[j]], add=True)

                @pl.when(j + NBUF < CHUNKS_PER_TILE)
                def _():
                    _load(j + NBUF, b).start()
        return carry

    lax.fori_loop(0, (CHUNKS_PER_TILE + NBUF - 1) // NBUF, _hyper, 0)

    plsc.subcore_barrier()

    for k in range(8):
        idx = s + 16 * k

        @pl.when(idx < ROW_CHUNKS)
        def _():
            row0 = pl.multiple_of(idx * CHUNK, CHUNK)
            pltpu.sync_copy(
                accum.at[pl.ds(row0, CHUNK)],
                out_hbm.at[c, pl.ds(row0, CHUNK)],
            )


def _scatter_stage(xm, dst3d):
    mesh = plsc.VectorSubcoreMesh(core_axis_name="c", subcore_axis_name="s")
    f = pl.kernel(
        _scatter_body,
        out_type=jax.ShapeDtypeStruct((2, N, HALF), jnp.float32),
        mesh=mesh,
        scratch_types=[
            pltpu.VMEM((CHUNKS_PER_TILE, CHUNK), jnp.int32),
            pltpu.VMEM((NBUF, CHUNK, HALF), jnp.float32),
            pltpu.VMEM_SHARED((N, HALF), jnp.float32),
            pltpu.SemaphoreType.DMA((NBUF,)),
        ],
    )
    return f(xm, dst3d)


def _mlp_body(xe_ref, w1_ref, wres_ref, wout_ref, out_ref):
    z = (jnp.dot(xe_ref[0], w1_ref[:HALF, :], preferred_element_type=jnp.float32)
         + jnp.dot(xe_ref[1], w1_ref[HALF:, :], preferred_element_type=jnp.float32))
    h = _silu(z)
    for i in range(N_HIDDEN):
        t = _silu(jnp.dot(h, wres_ref[i, 0], preferred_element_type=jnp.float32))
        t = _silu(jnp.dot(t, wres_ref[i, 1], preferred_element_type=jnp.float32))
        h = (h + t) * INV_SQRT2
    out_ref[...] = jnp.dot(h, wout_ref[...], preferred_element_type=jnp.float32)


def _mlp_stage(xe, W1, W_res, W_out):
    blk = 1000
    grid = N // blk
    return pl.pallas_call(
        _mlp_body,
        grid=(grid,),
        in_specs=[
            pl.BlockSpec((2, blk, HALF), lambda i: (0, i, 0)),
            pl.BlockSpec((D, D), lambda i: (0, 0)),
            pl.BlockSpec((N_HIDDEN, 2, D, D), lambda i: (0, 0, 0, 0)),
            pl.BlockSpec((D, 1), lambda i: (0, 0)),
        ],
        out_specs=pl.BlockSpec((blk, 1), lambda i: (i, 0)),
        out_shape=jax.ShapeDtypeStruct((N, 1), jnp.float32),
    )(xe, W1, W_res, W_out)


def kernel(x, edge_attr, edge_index, rbf, y, W_rbf_out, W_dense_rbf, W1, W_res, W_out):
    xm = _edge_stage(rbf.T, edge_attr, W_rbf_out, W_dense_rbf)
    dst3d = edge_index[1].reshape(16, CHUNKS_PER_TILE, CHUNK)
    xe = _scatter_stage(xm, dst3d)
    e_out = _mlp_stage(xe, W1, W_res, W_out)
    return (e_out, y)
